# Initial kernel scaffold; baseline (speedup 1.0000x reference)
import functools
import jax
import jax.numpy as jnp
from jax import lax
from jax.experimental import pallas as pl
from jax.experimental.pallas import tpu as pltpu
from jax.experimental.pallas import tpu_sc as plsc

_NC = 2
_NS = 16
_NW = _NC * _NS

_CHUNK = 1024


def _gather_body(idx_hbm, table_hbm, out_hbm, idx_v, rows_v, sem):
    wid = lax.axis_index("s") * _NC + lax.axis_index("c")
    n_total = idx_hbm.shape[0]
    b_per_w = n_total // _NW
    base = wid * b_per_w
    n_chunks = b_per_w // _CHUNK

    def body(i, carry):
        off = base + i * _CHUNK
        pltpu.sync_copy(idx_hbm.at[pl.ds(off, _CHUNK)], idx_v)
        pltpu.async_copy(table_hbm.at[idx_v], rows_v, sem).wait()
        pltpu.sync_copy(rows_v, out_hbm.at[pl.ds(off, _CHUNK)])
        return carry

    lax.fori_loop(0, n_chunks, body, 0)


def kernel(indic, table):
    bsz, fields = indic.shape
    n_rows, dim = table.shape
    n_total = bsz * fields
    idx = indic.reshape(n_total).astype(jnp.int32)

    gather = functools.partial(
        pl.kernel,
        out_type=jax.ShapeDtypeStruct((n_total, dim), jnp.float32),
        scratch_types=[
            pltpu.VMEM((_CHUNK,), jnp.int32),
            pltpu.VMEM((_CHUNK, dim), jnp.float32),
            pltpu.SemaphoreType.DMA,
        ],
        mesh=plsc.VectorSubcoreMesh(core_axis_name="c", subcore_axis_name="s"),
    )(_gather_body)

    out = gather(idx, table)
    return out.reshape(bsz, fields, dim)


# speedup vs baseline: 1.5546x; 1.5546x over previous
import functools
import jax
import jax.numpy as jnp
from jax import lax
from jax.experimental import pallas as pl
from jax.experimental.pallas import tpu as pltpu
from jax.experimental.pallas import tpu_sc as plsc

_NC = 2
_NS = 16
_NW = _NC * _NS

_CHUNK = 1024


def _gather_body(idx_hbm, table_hbm, out_hbm, idx_v, rows_v, sem):
    wid = lax.axis_index("s") * _NC + lax.axis_index("c")
    n_total = idx_hbm.shape[0]
    b_per_w = n_total // _NW
    base = wid * b_per_w
    n_chunks = b_per_w // _CHUNK

    def body(i, carry):
        off = base + i * _CHUNK
        pltpu.sync_copy(idx_hbm.at[pl.ds(off, _CHUNK)], idx_v)
        pltpu.async_copy(table_hbm.at[idx_v], rows_v, sem).wait()
        pltpu.sync_copy(rows_v, out_hbm.at[pl.ds(off, _CHUNK)])
        return carry

    lax.fori_loop(0, n_chunks, body, 0)


def kernel(indic, table):
    bsz, fields = indic.shape
    n_rows, dim = table.shape
    n_total = bsz * fields
    idx = indic.reshape(n_total).astype(jnp.int32)

    gather = functools.partial(
        pl.kernel,
        out_type=jax.ShapeDtypeStruct((n_total, dim), jnp.float32),
        scratch_types=[
            pltpu.VMEM((_CHUNK,), jnp.int32),
            pltpu.VMEM((_CHUNK, dim), jnp.float32),
            pltpu.SemaphoreType.DMA,
        ],
        mesh=plsc.VectorSubcoreMesh(core_axis_name="c", subcore_axis_name="s"),
        compiler_params=pltpu.CompilerParams(use_tc_tiling_on_sc=False),
    )(_gather_body)

    out = gather(idx, table)
    return out.reshape(bsz, fields, dim)


# speedup vs baseline: 1.5729x; 1.0117x over previous
import functools
import jax
import jax.numpy as jnp
from jax import lax
from jax.experimental import pallas as pl
from jax.experimental.pallas import tpu as pltpu
from jax.experimental.pallas import tpu_sc as plsc

_NC = 2
_NS = 16
_NW = _NC * _NS

_CHUNK = 832
_NBUF = 4
_DEPTH = 3


def _gather_body(idx_hbm, table_hbm, out_hbm, idx_v, rows_v, *sems):
    sem_g = sems[:_NBUF]
    sem_o = sems[_NBUF:]
    wid = lax.axis_index("s") * _NC + lax.axis_index("c")
    n_chunks = idx_hbm.shape[1]
    b_per_w = n_chunks * _CHUNK
    base = wid * b_per_w

    pltpu.sync_copy(idx_hbm.at[wid], idx_v)

    def out_slice(j):
        return out_hbm.at[pl.ds(base + j * _CHUNK, _CHUNK)]

    def start_gather(i):
        b = i % _NBUF
        pltpu.async_copy(table_hbm.at[idx_v.at[i]], rows_v.at[b], sem_g[b])

    def drain_gather_start_write(j):
        b = j % _NBUF
        pltpu.make_async_copy(table_hbm.at[idx_v.at[j]], rows_v.at[b], sem_g[b]).wait()
        pltpu.async_copy(rows_v.at[b], out_slice(j), sem_o[b])

    def wait_write(j):
        b = j % _NBUF
        pltpu.make_async_copy(rows_v.at[b], out_slice(j), sem_o[b]).wait()

    for i in range(n_chunks):
        if i >= _NBUF:
            wait_write(i - _NBUF)
        start_gather(i)
        if i >= _DEPTH - 1:
            drain_gather_start_write(i - (_DEPTH - 1))
    for j in range(n_chunks - (_DEPTH - 1), n_chunks):
        drain_gather_start_write(j)
    for j in range(max(0, n_chunks - _NBUF), n_chunks):
        wait_write(j)


def kernel(indic, table):
    bsz, fields = indic.shape
    n_rows, dim = table.shape
    n_total = bsz * fields
    idx = indic.reshape(n_total).astype(jnp.int32)

    grain = _NW * _CHUNK
    n_pad = (-n_total) % grain
    if n_pad:
        idx = jnp.concatenate([idx, jnp.zeros((n_pad,), jnp.int32)])
    n_chunks = (n_total + n_pad) // grain
    idx3 = idx.reshape(_NW, n_chunks, _CHUNK)

    gather = functools.partial(
        pl.kernel,
        out_type=jax.ShapeDtypeStruct((n_total + n_pad, dim), jnp.float32),
        scratch_types=[
            pltpu.VMEM((n_chunks, _CHUNK), jnp.int32),
            pltpu.VMEM((_NBUF, _CHUNK, dim), jnp.float32),
        ] + [pltpu.SemaphoreType.DMA] * (2 * _NBUF),
        mesh=plsc.VectorSubcoreMesh(core_axis_name="c", subcore_axis_name="s"),
        compiler_params=pltpu.CompilerParams(use_tc_tiling_on_sc=False),
    )(_gather_body)

    out = gather(idx3, table)
    if n_pad:
        out = out[:n_total]
    return out.reshape(bsz, fields, dim)
